# baseline (speedup 1.0000x reference)
import functools

import jax
import jax.numpy as jnp
from jax import lax
from jax.experimental import pallas as pl
from jax.experimental.pallas import tpu as pltpu
from jax.experimental.pallas import tpu_sc as plsc

NUM_CLASSES = 100000
FEAT = 64
BATCH = 16384
NUM_CORES = 2
NUM_SUBCORES = 16
LANES = 16
NW = NUM_CORES * NUM_SUBCORES
FPW = FEAT // NW
CHUNK = 2048
NCHUNKS = BATCH // CHUNK
GROUPS = CHUNK // LANES


def _sc_partials(xt, labels, cent):
    mesh = plsc.VectorSubcoreMesh(core_axis_name="c", subcore_axis_name="s")

    @functools.partial(
        pl.kernel,
        mesh=mesh,
        out_type=jax.ShapeDtypeStruct((NW, BATCH), jnp.float32),
        compiler_params=pltpu.CompilerParams(
            needs_layout_passes=False, use_tc_tiling_on_sc=True
        ),
        scratch_types=[
            pltpu.VMEM((NUM_CLASSES,), jnp.float32),
            pltpu.VMEM((2, CHUNK), jnp.int32),
            pltpu.VMEM((2, CHUNK), jnp.float32),
            pltpu.VMEM((BATCH,), jnp.float32),
            pltpu.SemaphoreType.DMA,
            pltpu.SemaphoreType.DMA,
            pltpu.SemaphoreType.DMA,
        ],
    )
    def k(xt_hbm, lab_hbm, cen_hbm, out_hbm, tab_v, lab_v, x_v, acc_v,
          sem_t, sem0, sem1):
        wid = lax.axis_index("s") * NUM_CORES + lax.axis_index("c")
        sems = (sem0, sem1)

        for p in range(FPW):
            f = wid + p * NW
            tab_cp = pltpu.async_copy(cen_hbm.at[f], tab_v, sem_t)
            pending = [
                pltpu.async_copy(
                    lab_hbm.at[pl.ds(0, CHUNK)], lab_v.at[0], sems[0]
                ),
                pltpu.async_copy(
                    xt_hbm.at[f, pl.ds(0, CHUNK)], x_v.at[0], sems[0]
                ),
            ]
            tab_cp.wait()
            for ch in range(NCHUNKS):
                buf = ch % 2
                nbuf = (ch + 1) % 2
                if ch + 1 < NCHUNKS:
                    nxt = [
                        pltpu.async_copy(
                            lab_hbm.at[pl.ds((ch + 1) * CHUNK, CHUNK)],
                            lab_v.at[nbuf], sems[nbuf],
                        ),
                        pltpu.async_copy(
                            xt_hbm.at[f, pl.ds((ch + 1) * CHUNK, CHUNK)],
                            x_v.at[nbuf], sems[nbuf],
                        ),
                    ]
                else:
                    nxt = []
                for cp in pending:
                    cp.wait()
                pending = nxt

                @plsc.parallel_loop(0, GROUPS, unroll=16)
                def group_body(g, ch=ch, p=p, buf=buf):
                    off = g * LANES
                    idx = lab_v[buf, pl.ds(off, LANES)]
                    cg = plsc.load_gather(tab_v, [idx])
                    xv = x_v[buf, pl.ds(off, LANES)]
                    d = xv - cg
                    d2 = d * d
                    aoff = ch * CHUNK + off
                    if p == 0:
                        acc_v[pl.ds(aoff, LANES)] = d2
                    else:
                        plsc.addupdate(acc_v.at[pl.ds(aoff, LANES)], d2)
        pltpu.sync_copy(acc_v, out_hbm.at[wid])

    return k(xt, labels, cent)


def _tc_reduce(partials):
    def body(p_ref, o_ref):
        dist = jnp.sum(p_ref[...], axis=0)
        dist = jnp.minimum(jnp.maximum(dist, 1e-12), 1e12)
        o_ref[0, 0] = jnp.sum(dist) * (1.0 / BATCH)

    return pl.pallas_call(
        body,
        out_shape=jax.ShapeDtypeStruct((1, 1), jnp.float32),
        out_specs=pl.BlockSpec(memory_space=pltpu.SMEM),
    )(partials)


def kernel(x, labels, centers):
    partials = _sc_partials(x.T, labels.astype(jnp.int32), centers.T)
    return _tc_reduce(partials)[0, 0]


# speedup vs baseline: 1.0452x; 1.0452x over previous
import functools

import jax
import jax.numpy as jnp
from jax import lax
from jax.experimental import pallas as pl
from jax.experimental.pallas import tpu as pltpu
from jax.experimental.pallas import tpu_sc as plsc

NUM_CLASSES = 100000
FEAT = 64
BATCH = 16384
NUM_CORES = 2
NUM_SUBCORES = 16
LANES = 16
NW = NUM_CORES * NUM_SUBCORES
FPW = FEAT // NW
CHUNK = 2048
NCHUNKS = BATCH // CHUNK
GROUPS = CHUNK // LANES


def _sc_partials(xt, labels, cent):
    mesh = plsc.VectorSubcoreMesh(core_axis_name="c", subcore_axis_name="s")

    @functools.partial(
        pl.kernel,
        mesh=mesh,
        out_type=jax.ShapeDtypeStruct((NW, BATCH), jnp.float32),
        compiler_params=pltpu.CompilerParams(
            needs_layout_passes=False, use_tc_tiling_on_sc=True
        ),
        scratch_types=[
            pltpu.VMEM((NUM_CLASSES,), jnp.float32),
            pltpu.VMEM((2, CHUNK), jnp.int32),
            pltpu.VMEM((2, CHUNK), jnp.float32),
            pltpu.VMEM((BATCH,), jnp.float32),
            pltpu.SemaphoreType.DMA,
            pltpu.SemaphoreType.DMA,
            pltpu.SemaphoreType.DMA,
        ],
    )
    def k(xt_hbm, lab_hbm, cen_hbm, out_hbm, tab_v, lab_v, x_v, acc_v,
          sem_t, sem0, sem1):
        wid = lax.axis_index("s") * NUM_CORES + lax.axis_index("c")
        sems = (sem0, sem1)

        for p in range(FPW):
            f = wid + p * NW
            tab_cp = pltpu.async_copy(cen_hbm.at[f], tab_v, sem_t)
            pending = [
                pltpu.async_copy(
                    lab_hbm.at[pl.ds(0, CHUNK)], lab_v.at[0], sems[0]
                ),
                pltpu.async_copy(
                    xt_hbm.at[f, pl.ds(0, CHUNK)], x_v.at[0], sems[0]
                ),
            ]
            tab_cp.wait()
            for ch in range(NCHUNKS):
                buf = ch % 2
                nbuf = (ch + 1) % 2
                if ch + 1 < NCHUNKS:
                    nxt = [
                        pltpu.async_copy(
                            lab_hbm.at[pl.ds((ch + 1) * CHUNK, CHUNK)],
                            lab_v.at[nbuf], sems[nbuf],
                        ),
                        pltpu.async_copy(
                            xt_hbm.at[f, pl.ds((ch + 1) * CHUNK, CHUNK)],
                            x_v.at[nbuf], sems[nbuf],
                        ),
                    ]
                else:
                    nxt = []
                for cp in pending:
                    cp.wait()
                pending = nxt

                @plsc.parallel_loop(0, GROUPS, unroll=8)
                def group_body(g, ch=ch, p=p, buf=buf):
                    off = g * LANES
                    idx = lab_v[buf, pl.ds(off, LANES)]
                    cg = plsc.load_gather(tab_v, [idx])
                    xv = x_v[buf, pl.ds(off, LANES)]
                    d = xv - cg
                    d2 = d * d
                    aoff = ch * CHUNK + off
                    if p == 0:
                        acc_v[pl.ds(aoff, LANES)] = d2
                    else:
                        plsc.addupdate(acc_v.at[pl.ds(aoff, LANES)], d2)
        pltpu.sync_copy(acc_v, out_hbm.at[wid])

    return k(xt, labels, cent)


def _tc_reduce(partials):
    def body(p_ref, o_ref):
        dist = jnp.sum(p_ref[...], axis=0)
        dist = jnp.minimum(jnp.maximum(dist, 1e-12), 1e12)
        o_ref[0, 0] = jnp.sum(dist) * (1.0 / BATCH)

    return pl.pallas_call(
        body,
        out_shape=jax.ShapeDtypeStruct((1, 1), jnp.float32),
        out_specs=pl.BlockSpec(memory_space=pltpu.SMEM),
    )(partials)


def kernel(x, labels, centers):
    partials = _sc_partials(x.T, labels.astype(jnp.int32), centers.T)
    return _tc_reduce(partials)[0, 0]


# speedup vs baseline: 1.0632x; 1.0173x over previous
import functools

import jax
import jax.numpy as jnp
from jax import lax
from jax.experimental import pallas as pl
from jax.experimental.pallas import tpu as pltpu
from jax.experimental.pallas import tpu_sc as plsc

NUM_CLASSES = 100000
FEAT = 64
BATCH = 16384
NUM_CORES = 2
NUM_SUBCORES = 16
LANES = 16
NW = NUM_CORES * NUM_SUBCORES
FPW = FEAT // NW
CHUNK = 2048
NCHUNKS = BATCH // CHUNK
GROUPS = CHUNK // LANES


def _sc_partials(xt, labels, cent):
    mesh = plsc.VectorSubcoreMesh(core_axis_name="c", subcore_axis_name="s")

    @functools.partial(
        pl.kernel,
        mesh=mesh,
        out_type=jax.ShapeDtypeStruct(
            (NUM_CORES, BATCH // 128, 128), jnp.float32
        ),
        compiler_params=pltpu.CompilerParams(
            needs_layout_passes=False, use_tc_tiling_on_sc=True
        ),
        scratch_types=[
            pltpu.VMEM((NUM_CLASSES,), jnp.float32),
            pltpu.VMEM((2, CHUNK), jnp.int32),
            pltpu.VMEM((2, CHUNK), jnp.float32),
            pltpu.VMEM((BATCH // 128, 128), jnp.float32),
            pltpu.VMEM_SHARED((BATCH // 128, 128), jnp.float32),
            pltpu.VMEM((BATCH // 128,), jnp.int32),
            pltpu.SemaphoreType.DMA,
            pltpu.SemaphoreType.DMA,
            pltpu.SemaphoreType.DMA,
        ],
    )
    def k(xt_hbm, lab_hbm, cen_hbm, out_hbm, tab_v, lab_v, x_v, acc_v,
          shared_v, rowidx_v, sem_t, sem0, sem1):
        sid = lax.axis_index("s")
        core = lax.axis_index("c")
        wid = sid * NUM_CORES + core
        sems = (sem0, sem1)

        lane_iota0 = lax.iota(jnp.int32, LANES)
        for gg in range(BATCH // 128 // LANES):
            rowidx_v[pl.ds(gg * LANES, LANES)] = gg * LANES + lane_iota0

        for p in range(FPW):
            f = wid + p * NW
            tab_cp = pltpu.async_copy(cen_hbm.at[f], tab_v, sem_t)
            pending = [
                pltpu.async_copy(
                    lab_hbm.at[pl.ds(0, CHUNK)], lab_v.at[0], sems[0]
                ),
                pltpu.async_copy(
                    xt_hbm.at[f, pl.ds(0, CHUNK)], x_v.at[0], sems[0]
                ),
            ]
            tab_cp.wait()
            for ch in range(NCHUNKS):
                buf = ch % 2
                nbuf = (ch + 1) % 2
                if ch + 1 < NCHUNKS:
                    nxt = [
                        pltpu.async_copy(
                            lab_hbm.at[pl.ds((ch + 1) * CHUNK, CHUNK)],
                            lab_v.at[nbuf], sems[nbuf],
                        ),
                        pltpu.async_copy(
                            xt_hbm.at[f, pl.ds((ch + 1) * CHUNK, CHUNK)],
                            x_v.at[nbuf], sems[nbuf],
                        ),
                    ]
                else:
                    nxt = []
                for cp in pending:
                    cp.wait()
                pending = nxt

                @plsc.parallel_loop(0, GROUPS, unroll=8)
                def group_body(g, ch=ch, p=p, buf=buf):
                    off = g * LANES
                    idx = lab_v[buf, pl.ds(off, LANES)]
                    cg = plsc.load_gather(tab_v, [idx])
                    xv = x_v[buf, pl.ds(off, LANES)]
                    d = xv - cg
                    d2 = d * d
                    aoff = ch * CHUNK + off
                    ar = lax.shift_right_logical(aoff, 7)
                    ac = lax.bitwise_and(aoff, 127)
                    if p == 0:
                        acc_v[ar, pl.ds(ac, LANES)] = d2
                    else:
                        plsc.addupdate(acc_v.at[ar, pl.ds(ac, LANES)], d2)
        @pl.when(sid == 0)
        def _():
            pltpu.sync_copy(acc_v, shared_v)

        plsc.subcore_barrier()

        @pl.when(sid != 0)
        def _():
            pltpu.sync_copy(acc_v, shared_v.at[rowidx_v], add=True)

        plsc.subcore_barrier()

        @pl.when(sid == 0)
        def _():
            pltpu.sync_copy(shared_v, out_hbm.at[core])

    return k(xt, labels, cent)


def _tc_reduce(partials):
    def body(p_ref, o_ref):
        dist = jnp.sum(p_ref[...], axis=0)
        dist = jnp.minimum(jnp.maximum(dist, 1e-12), 1e12)
        o_ref[0, 0] = jnp.sum(dist) * (1.0 / BATCH)

    return pl.pallas_call(
        body,
        out_shape=jax.ShapeDtypeStruct((1, 1), jnp.float32),
        out_specs=pl.BlockSpec(memory_space=pltpu.SMEM),
    )(partials)


def kernel(x, labels, centers):
    partials = _sc_partials(x.T, labels.astype(jnp.int32), centers.T)
    return _tc_reduce(partials)[0, 0]
